# Initial kernel scaffold; baseline (speedup 1.0000x reference)
import jax
import jax.numpy as jnp
from jax.experimental import pallas as pl


def kernel(wt_pos, mt_pos, wt_x, mt_x, wt_batch, mt_batch, edge_index, W_in, W1, b1, W2, b2, W_sh, W_out, W_lin, b_lin):
    raise NotImplementedError("write your pallas kernel here")


# speedup vs baseline: 1.8286x; 1.8286x over previous
import functools

import jax
import jax.numpy as jnp
import numpy as np
from jax import lax
from jax.experimental import pallas as pl
from jax.experimental.pallas import tpu as pltpu
from jax.experimental.pallas import tpu_sc as plsc

N = 10000
E = 320000
H = 72
HP = 80
XP = 32
PP = 16
G = 64
SH_DIM = 16
N_BASIS = 10
MAX_RADIUS = 20.0
STEP = MAX_RADIUS / (N_BASIS - 1)
INV_SQRT_NEI = float(1.0 / np.sqrt(32.0))

CH = 512
IR = CH // 128
NCH = E // CH
NC, NS = 2, 16
NW = NC * NS
BE = 2560


def _sc_gather_body(pos_hbm, xp_hbm, src2_hbm, dst2_hbm,
                    ps_out, pd_out, xs_out,
                    sbuf, dbuf, ps, pd, xs, sem):
    c = lax.axis_index("c")
    s = lax.axis_index("s")
    wid = s * NC + c

    def body(i, carry):
        ci = wid + i * NW
        pltpu.sync_copy(src2_hbm.at[pl.ds(ci * IR, IR)], sbuf)
        pltpu.sync_copy(dst2_hbm.at[pl.ds(ci * IR, IR)], dbuf)
        cps = []
        for j in range(IR):
            sl = pl.ds(j * 128, 128)
            cps.append(pltpu.async_copy(pos_hbm.at[sbuf.at[j]], ps.at[sl], sem))
            cps.append(pltpu.async_copy(pos_hbm.at[dbuf.at[j]], pd.at[sl], sem))
            cps.append(pltpu.async_copy(xp_hbm.at[sbuf.at[j]], xs.at[sl], sem))
        for cp in cps:
            cp.wait()
        e0 = ci * CH
        pltpu.sync_copy(ps, ps_out.at[pl.ds(e0, CH)])
        pltpu.sync_copy(pd, pd_out.at[pl.ds(e0, CH)])
        pltpu.sync_copy(xs, xs_out.at[pl.ds(e0, CH)])
        return carry

    nmine = (NCH - wid + NW - 1) // NW
    lax.fori_loop(0, nmine, body, 0)


@functools.cache
def _make_sc_gather():
    return pl.kernel(
        _sc_gather_body,
        out_type=(jax.ShapeDtypeStruct((E, PP), jnp.float32),
                  jax.ShapeDtypeStruct((E, PP), jnp.float32),
                  jax.ShapeDtypeStruct((E, XP), jnp.float32)),
        mesh=plsc.VectorSubcoreMesh(core_axis_name="c", subcore_axis_name="s"),
        scratch_types=[pltpu.VMEM((IR, 128), jnp.int32),
                       pltpu.VMEM((IR, 128), jnp.int32),
                       pltpu.VMEM((CH, PP), jnp.float32),
                       pltpu.VMEM((CH, PP), jnp.float32),
                       pltpu.VMEM((CH, XP), jnp.float32),
                       pltpu.SemaphoreType.DMA],
        compiler_params=pltpu.CompilerParams(use_tc_tiling_on_sc=False),
    )


def _sc_scatter_body(msg_hbm, dst2_hbm, zeros_hbm, part_out,
                     dbuf, msgb, acc):
    c = lax.axis_index("c")
    s = lax.axis_index("s")
    wid = s * NC + c

    @pl.when(s == 0)
    def _init():
        pltpu.sync_copy(zeros_hbm, acc)

    plsc.subcore_barrier()

    def body(i, carry):
        ci = wid + i * NW
        pltpu.sync_copy(dst2_hbm.at[pl.ds(ci * IR, IR)], dbuf)
        pltpu.sync_copy(msg_hbm.at[pl.ds(ci * CH, CH)], msgb)
        for j in range(IR):
            pltpu.sync_copy(msgb.at[pl.ds(j * 128, 128)],
                            acc.at[dbuf.at[j]], add=True)
        return carry

    nmine = (NCH - wid + NW - 1) // NW
    lax.fori_loop(0, nmine, body, 0)

    plsc.subcore_barrier()
    rpt = N // NS
    pltpu.sync_copy(acc.at[pl.ds(s * rpt, rpt)],
                    part_out.at[pl.ds(c * N + s * rpt, rpt)])


@functools.cache
def _make_sc_scatter():
    return pl.kernel(
        _sc_scatter_body,
        out_type=jax.ShapeDtypeStruct((2 * N, HP), jnp.float32),
        mesh=plsc.VectorSubcoreMesh(core_axis_name="c", subcore_axis_name="s"),
        scratch_types=[pltpu.VMEM((IR, 128), jnp.int32),
                       pltpu.VMEM((CH, HP), jnp.float32),
                       pltpu.VMEM_SHARED((N, HP), jnp.float32)],
        compiler_params=pltpu.CompilerParams(use_tc_tiling_on_sc=False),
    )


def _tc_edge_body(ps_ref, pd_ref, xs_ref, Win_ref, W1_ref, b1_ref, W2_ref,
                  b2_ref, Wsh_ref, msg_ref):
    ev = ps_ref[...] - pd_ref[...]
    x = ev[:, 0:1]
    y = ev[:, 1:2]
    z = ev[:, 2:3]
    r2 = x * x + y * y + z * z + 1e-9
    r = jnp.sqrt(r2)
    inv = 1.0 / r
    ux = x * inv
    uy = y * inv
    uz = z * inv
    s3, s5, s15 = np.sqrt(3.0), np.sqrt(5.0), np.sqrt(15.0)
    sh = jnp.concatenate([
        jnp.ones_like(ux),
        s3 * ux, s3 * uy, s3 * uz,
        s15 * ux * uy, s15 * uy * uz, 0.5 * s5 * (3 * uz * uz - 1.0),
        s15 * ux * uz, 0.5 * s15 * (ux * ux - uy * uy),
        np.sqrt(35.0 / 8.0) * uy * (3 * ux * ux - uy * uy),
        np.sqrt(105.0) * ux * uy * uz,
        np.sqrt(21.0 / 8.0) * uy * (5 * uz * uz - 1.0),
        0.5 * np.sqrt(7.0) * (5 * uz * uz * uz - 3 * uz),
        np.sqrt(21.0 / 8.0) * ux * (5 * uz * uz - 1.0),
        0.5 * np.sqrt(105.0) * (ux * ux - uy * uy) * uz,
        np.sqrt(35.0 / 8.0) * ux * (ux * ux - uy * uy),
    ], axis=1)

    cent = lax.broadcasted_iota(
        jnp.int32, (BE, N_BASIS), 1).astype(jnp.float32) * STEP
    d = (r - cent) * (1.0 / STEP)
    emb = jnp.exp(-d * d) * np.sqrt(float(N_BASIS))

    z1 = jnp.maximum(
        jnp.dot(emb, W1_ref[...], preferred_element_type=jnp.float32)
        + b1_ref[...], 0.0)
    w = jnp.dot(z1, W2_ref[...],
                preferred_element_type=jnp.float32) + b2_ref[...]
    shw = jnp.dot(sh, Wsh_ref[...], preferred_element_type=jnp.float32)

    u = jnp.clip(r * (1.0 / MAX_RADIUS), 0.0, 1.0)
    v = np.pi * (u - 0.5)
    v2 = v * v
    sinv = v * (1.0 + v2 * (-1.0 / 6.0 + v2 * (1.0 / 120.0 + v2 * (
        -1.0 / 5040.0 + v2 * (1.0 / 362880.0)))))
    cut = 0.5 * (1.0 - sinv)

    hs = jnp.dot(xs_ref[...], Win_ref[...],
                 preferred_element_type=jnp.float32)
    msg_ref[...] = hs * w * shw * cut


_tc_edge = pl.pallas_call(
    _tc_edge_body,
    grid=(E // BE,),
    in_specs=[
        pl.BlockSpec((BE, PP), lambda i: (i, 0)),
        pl.BlockSpec((BE, PP), lambda i: (i, 0)),
        pl.BlockSpec((BE, XP), lambda i: (i, 0)),
        pl.BlockSpec((XP, HP), lambda i: (0, 0)),
        pl.BlockSpec((N_BASIS, 128), lambda i: (0, 0)),
        pl.BlockSpec((1, 128), lambda i: (0, 0)),
        pl.BlockSpec((128, HP), lambda i: (0, 0)),
        pl.BlockSpec((1, HP), lambda i: (0, 0)),
        pl.BlockSpec((SH_DIM, HP), lambda i: (0, 0)),
    ],
    out_specs=pl.BlockSpec((BE, HP), lambda i: (i, 0)),
    out_shape=jax.ShapeDtypeStruct((E, HP), jnp.float32),
)


def _tc_final_body(xp_ref, part_ref, batch_ref, Win_ref, Wout_ref, Wlin_ref,
                   blin_ref, out_ref):
    h = jnp.dot(xp_ref[...], Win_ref[...],
                preferred_element_type=jnp.float32)
    part = part_ref[...]
    h2 = h + (part[0:N] + part[N:2 * N]) * INV_SQRT_NEI
    gids = lax.broadcasted_iota(jnp.int32, (G, N), 0).astype(jnp.float32)
    oh = jnp.where(gids == batch_ref[...], 1.0, 0.0)
    sums = jnp.dot(oh, h2, preferred_element_type=jnp.float32)
    counts = jnp.sum(oh, axis=1, keepdims=True)
    pooled = sums / jnp.maximum(counts, 1.0)
    wc = jnp.dot(Wout_ref[...], Wlin_ref[...],
                 preferred_element_type=jnp.float32)
    out_ref[...] = jnp.dot(pooled, wc,
                           preferred_element_type=jnp.float32) + blin_ref[...]


_tc_final = pl.pallas_call(
    _tc_final_body,
    out_shape=jax.ShapeDtypeStruct((G, 128), jnp.float32),
)


def kernel(wt_pos, mt_pos, wt_x, mt_x, wt_batch, mt_batch, edge_index,
           W_in, W1, b1, W2, b2, W_sh, W_out, W_lin, b_lin):
    f32 = jnp.float32
    pos_p = jnp.pad(jnp.concatenate([wt_pos, mt_pos], 0),
                    ((0, 0), (0, PP - 3)))
    x_p = jnp.pad(jnp.concatenate([wt_x, mt_x], 0), ((0, 0), (0, XP - 25)))
    batch = jnp.concatenate([wt_batch, mt_batch]).astype(f32).reshape(1, N)
    src2 = edge_index[0].astype(jnp.int32).reshape(E // 128, 128)
    dst2 = edge_index[1].astype(jnp.int32).reshape(E // 128, 128)

    Win_p = jnp.pad(W_in, ((0, XP - 25), (0, HP - H)))
    W2_p = jnp.pad(W2, ((0, 0), (0, HP - H)))
    b1_r = b1.reshape(1, 128)
    b2_r = jnp.pad(b2, (0, HP - H)).reshape(1, HP)
    Wsh_p = jnp.pad(W_sh, ((0, 0), (0, HP - H)))
    Wout_p = jnp.pad(W_out, ((0, HP - H), (0, 0)))
    Wlin_p = jnp.pad(W_lin, ((0, 0), (0, 128 - 2)))
    blin_p = jnp.pad(b_lin, (0, 128 - 2)).reshape(1, 128)
    zeros_nh = jnp.zeros((N, HP), f32)

    ps, pd, xs = _make_sc_gather()(pos_p, x_p, src2, dst2)
    msg = _tc_edge(ps, pd, xs, Win_p, W1, b1_r, W2_p, b2_r, Wsh_p)
    part = _make_sc_scatter()(msg, dst2, zeros_nh)
    outm = _tc_final(x_p, part, batch, Win_p, Wout_p, Wlin_p, blin_p)
    o = outm[:50, :2]
    return (o[:, 0], o[:, 1])


# speedup vs baseline: 2.2911x; 1.2530x over previous
import functools

import jax
import jax.numpy as jnp
import numpy as np
from jax import lax
from jax.experimental import pallas as pl
from jax.experimental.pallas import tpu as pltpu
from jax.experimental.pallas import tpu_sc as plsc

N = 10000
E = 320000
H = 72
HP = 128
XP = 32
PP = 16
G = 64
SH_DIM = 16
N_BASIS = 10
MAX_RADIUS = 20.0
STEP = MAX_RADIUS / (N_BASIS - 1)
INV_SQRT_NEI = float(1.0 / np.sqrt(32.0))

HA = 80
CH = 512
IR = CH // 128
NCH = E // CH
NC, NS = 2, 16
NW = NC * NS
BE = 2560


def _sc_gather_body(pos_hbm, xp_hbm, src2_hbm, dst2_hbm, pk_out,
                    sbuf, dbuf, ps, pd, xs, sem):
    c = lax.axis_index("c")
    s = lax.axis_index("s")
    wid = s * NC + c

    def body(i, carry):
        ci = wid + i * NW
        pltpu.sync_copy(src2_hbm.at[pl.ds(ci * IR, IR)], sbuf)
        pltpu.sync_copy(dst2_hbm.at[pl.ds(ci * IR, IR)], dbuf)
        cps = []
        for j in range(IR):
            sl = pl.ds(j * 128, 128)
            cps.append(pltpu.async_copy(pos_hbm.at[sbuf.at[j]], ps.at[sl], sem))
            cps.append(pltpu.async_copy(pos_hbm.at[dbuf.at[j]], pd.at[sl], sem))
            cps.append(pltpu.async_copy(xp_hbm.at[sbuf.at[j]], xs.at[sl], sem))
        for cp in cps:
            cp.wait()
        e0 = ci * CH
        rows = pl.ds(e0, CH)
        pltpu.sync_copy(ps, pk_out.at[rows, pl.ds(0, PP)])
        pltpu.sync_copy(pd, pk_out.at[rows, pl.ds(PP, PP)])
        pltpu.sync_copy(xs, pk_out.at[rows, pl.ds(2 * PP, XP)])
        return carry

    nmine = (NCH - wid + NW - 1) // NW
    lax.fori_loop(0, nmine, body, 0)


@functools.cache
def _make_sc_gather():
    return pl.kernel(
        _sc_gather_body,
        out_type=jax.ShapeDtypeStruct((E, 128), jnp.float32),
        mesh=plsc.VectorSubcoreMesh(core_axis_name="c", subcore_axis_name="s"),
        scratch_types=[pltpu.VMEM((IR, 128), jnp.int32),
                       pltpu.VMEM((IR, 128), jnp.int32),
                       pltpu.VMEM((CH, PP), jnp.float32),
                       pltpu.VMEM((CH, PP), jnp.float32),
                       pltpu.VMEM((CH, XP), jnp.float32),
                       pltpu.SemaphoreType.DMA],
        compiler_params=pltpu.CompilerParams(use_tc_tiling_on_sc=False),
    )


def _sc_scatter_body(msg_hbm, dst2_hbm, zeros_hbm, part_out,
                     dbuf, msgb, acc):
    c = lax.axis_index("c")
    s = lax.axis_index("s")
    wid = s * NC + c

    @pl.when(s == 0)
    def _init():
        pltpu.sync_copy(zeros_hbm, acc)

    plsc.subcore_barrier()

    def body(i, carry):
        ci = wid + i * NW
        pltpu.sync_copy(dst2_hbm.at[pl.ds(ci * IR, IR)], dbuf)
        pltpu.sync_copy(msg_hbm.at[pl.ds(ci * CH, CH), pl.ds(0, HA)], msgb)
        for j in range(IR):
            pltpu.sync_copy(msgb.at[pl.ds(j * 128, 128)],
                            acc.at[dbuf.at[j]], add=True)
        return carry

    nmine = (NCH - wid + NW - 1) // NW
    lax.fori_loop(0, nmine, body, 0)

    plsc.subcore_barrier()
    rpt = N // NS
    pltpu.sync_copy(acc.at[pl.ds(s * rpt, rpt)],
                    part_out.at[pl.ds(c * N + s * rpt, rpt)])


@functools.cache
def _make_sc_scatter():
    return pl.kernel(
        _sc_scatter_body,
        out_type=jax.ShapeDtypeStruct((2 * N, HA), jnp.float32),
        mesh=plsc.VectorSubcoreMesh(core_axis_name="c", subcore_axis_name="s"),
        scratch_types=[pltpu.VMEM((IR, 128), jnp.int32),
                       pltpu.VMEM((CH, HA), jnp.float32),
                       pltpu.VMEM_SHARED((N, HA), jnp.float32)],
        compiler_params=pltpu.CompilerParams(use_tc_tiling_on_sc=False),
    )


def _tc_edge_body(pk_ref, Win_ref, W1_ref, b1_ref, W2_ref,
                  b2_ref, Wsh_ref, msg_ref):
    pk = pk_ref[...]
    x = pk[:, 0:1] - pk[:, PP:PP + 1]
    y = pk[:, 1:2] - pk[:, PP + 1:PP + 2]
    z = pk[:, 2:3] - pk[:, PP + 2:PP + 3]
    r2 = x * x + y * y + z * z + 1e-9
    r = jnp.sqrt(r2)
    inv = 1.0 / r
    ux = x * inv
    uy = y * inv
    uz = z * inv
    s3, s5, s15 = np.sqrt(3.0), np.sqrt(5.0), np.sqrt(15.0)
    sh = jnp.concatenate([
        jnp.ones_like(ux),
        s3 * ux, s3 * uy, s3 * uz,
        s15 * ux * uy, s15 * uy * uz, 0.5 * s5 * (3 * uz * uz - 1.0),
        s15 * ux * uz, 0.5 * s15 * (ux * ux - uy * uy),
        np.sqrt(35.0 / 8.0) * uy * (3 * ux * ux - uy * uy),
        np.sqrt(105.0) * ux * uy * uz,
        np.sqrt(21.0 / 8.0) * uy * (5 * uz * uz - 1.0),
        0.5 * np.sqrt(7.0) * (5 * uz * uz * uz - 3 * uz),
        np.sqrt(21.0 / 8.0) * ux * (5 * uz * uz - 1.0),
        0.5 * np.sqrt(105.0) * (ux * ux - uy * uy) * uz,
        np.sqrt(35.0 / 8.0) * ux * (ux * ux - uy * uy),
    ], axis=1)

    cent = lax.broadcasted_iota(
        jnp.int32, (BE, N_BASIS), 1).astype(jnp.float32) * STEP
    d = (r - cent) * (1.0 / STEP)
    emb = jnp.exp(-d * d) * np.sqrt(float(N_BASIS))

    z1 = jnp.maximum(
        jnp.dot(emb, W1_ref[...], preferred_element_type=jnp.float32)
        + b1_ref[...], 0.0)
    w = jnp.dot(z1, W2_ref[...],
                preferred_element_type=jnp.float32) + b2_ref[...]
    shw = jnp.dot(sh, Wsh_ref[...], preferred_element_type=jnp.float32)

    u = jnp.clip(r * (1.0 / MAX_RADIUS), 0.0, 1.0)
    v = np.pi * (u - 0.5)
    v2 = v * v
    sinv = v * (1.0 + v2 * (-1.0 / 6.0 + v2 * (1.0 / 120.0 + v2 * (
        -1.0 / 5040.0 + v2 * (1.0 / 362880.0)))))
    cut = 0.5 * (1.0 - sinv)

    hs = jnp.dot(pk[:, 2 * PP:2 * PP + XP], Win_ref[...],
                 preferred_element_type=jnp.float32)
    msg_ref[...] = hs * w * shw * cut


_tc_edge = pl.pallas_call(
    _tc_edge_body,
    grid=(E // BE,),
    in_specs=[
        pl.BlockSpec((BE, 128), lambda i: (i, 0)),
        pl.BlockSpec((XP, HP), lambda i: (0, 0)),
        pl.BlockSpec((N_BASIS, 128), lambda i: (0, 0)),
        pl.BlockSpec((1, 128), lambda i: (0, 0)),
        pl.BlockSpec((128, HP), lambda i: (0, 0)),
        pl.BlockSpec((1, HP), lambda i: (0, 0)),
        pl.BlockSpec((SH_DIM, HP), lambda i: (0, 0)),
    ],
    out_specs=pl.BlockSpec((BE, HP), lambda i: (i, 0)),
    out_shape=jax.ShapeDtypeStruct((E, HP), jnp.float32),
)


def _tc_final_body(xp_ref, part_ref, batch_ref, Win_ref, Wout_ref, Wlin_ref,
                   blin_ref, out_ref):
    h = jnp.dot(xp_ref[...], Win_ref[...],
                preferred_element_type=jnp.float32)
    part = part_ref[...]
    agg = jnp.concatenate(
        [part[0:N] + part[N:2 * N],
         jnp.zeros((N, HP - HA), jnp.float32)], axis=1)
    h2 = h + agg * INV_SQRT_NEI
    gids = lax.broadcasted_iota(jnp.int32, (G, N), 0).astype(jnp.float32)
    oh = jnp.where(gids == batch_ref[...], 1.0, 0.0)
    sums = jnp.dot(oh, h2, preferred_element_type=jnp.float32)
    counts = jnp.sum(oh, axis=1, keepdims=True)
    pooled = sums / jnp.maximum(counts, 1.0)
    wc = jnp.dot(Wout_ref[...], Wlin_ref[...],
                 preferred_element_type=jnp.float32)
    out_ref[...] = jnp.dot(pooled, wc,
                           preferred_element_type=jnp.float32) + blin_ref[...]


_tc_final = pl.pallas_call(
    _tc_final_body,
    out_shape=jax.ShapeDtypeStruct((G, 128), jnp.float32),
)


def kernel(wt_pos, mt_pos, wt_x, mt_x, wt_batch, mt_batch, edge_index,
           W_in, W1, b1, W2, b2, W_sh, W_out, W_lin, b_lin):
    f32 = jnp.float32
    pos_p = jnp.pad(jnp.concatenate([wt_pos, mt_pos], 0),
                    ((0, 0), (0, PP - 3)))
    x_p = jnp.pad(jnp.concatenate([wt_x, mt_x], 0), ((0, 0), (0, XP - 25)))
    batch = jnp.concatenate([wt_batch, mt_batch]).astype(f32).reshape(1, N)
    src2 = edge_index[0].astype(jnp.int32).reshape(E // 128, 128)
    dst2 = edge_index[1].astype(jnp.int32).reshape(E // 128, 128)

    Win_p = jnp.pad(W_in, ((0, XP - 25), (0, HP - H)))
    W2_p = jnp.pad(W2, ((0, 0), (0, HP - H)))
    b1_r = b1.reshape(1, 128)
    b2_r = jnp.pad(b2, (0, HP - H)).reshape(1, HP)
    Wsh_p = jnp.pad(W_sh, ((0, 0), (0, HP - H)))
    Wout_p = jnp.pad(W_out, ((0, HP - H), (0, 0)))
    Wlin_p = jnp.pad(W_lin, ((0, 0), (0, 128 - 2)))
    blin_p = jnp.pad(b_lin, (0, 128 - 2)).reshape(1, 128)
    zeros_nh = jnp.zeros((N, HA), f32)

    pk = _make_sc_gather()(pos_p, x_p, src2, dst2)
    msg = _tc_edge(pk, Win_p, W1, b1_r, W2_p, b2_r, Wsh_p)
    part = _make_sc_scatter()(msg, dst2, zeros_nh)
    outm = _tc_final(x_p, part, batch, Win_p, Wout_p, Wlin_p, blin_p)
    o = outm[:50, :2]
    return (o[:, 0], o[:, 1])


# speedup vs baseline: 6.2797x; 2.7409x over previous
import functools

import jax
import jax.numpy as jnp
import numpy as np
from jax import lax
from jax.experimental import pallas as pl
from jax.experimental.pallas import tpu as pltpu
from jax.experimental.pallas import tpu_sc as plsc

N = 10000
E = 320000
H = 72
HP = 128
XP = 32
G = 64
SH_DIM = 16
N_BASIS = 10
MAX_RADIUS = 20.0
STEP = MAX_RADIUS / (N_BASIS - 1)
INV_SQRT_NEI = float(1.0 / np.sqrt(32.0))

HA = 80
CH = 512
IR = CH // 128
NCH = E // CH
NC, NS = 2, 16
NW = NC * NS
BE = 2560

C_EMB = 16
C_X = 32


def _rsqrt_newton(r2):
    i = plsc.bitcast(r2, jnp.int32)
    y = plsc.bitcast(jnp.int32(0x5F3759DF) - (i >> 1), jnp.float32)
    for _ in range(3):
        y = y * (1.5 - 0.5 * r2 * y * y)
    return y


def _sc_geo_body(pos4_hbm, xp_hbm, src1_hbm, src2_hbm, pk_out,
                 pos4_v, sbuf, dbuf, s2buf, geo, xs, sem):
    c = lax.axis_index("c")
    s = lax.axis_index("s")
    wid = s * NC + c
    pltpu.sync_copy(pos4_hbm, pos4_v)

    lane = lax.iota(jnp.int32, 16)
    s3, s5, s15 = np.sqrt(3.0), np.sqrt(5.0), np.sqrt(15.0)
    c35 = np.sqrt(35.0 / 8.0)
    c105 = np.sqrt(105.0)
    c21 = np.sqrt(21.0 / 8.0)
    c7 = 0.5 * np.sqrt(7.0)
    sq10 = np.sqrt(float(N_BASIS))

    def chunk(i, carry):
        ci = wid + i * NW
        e0 = ci * CH
        pltpu.sync_copy(src1_hbm.at[pl.ds(e0, CH)], sbuf)
        pltpu.sync_copy(src1_hbm.at[pl.ds(E + e0, CH)], dbuf)
        pltpu.sync_copy(src2_hbm.at[pl.ds(ci * IR, IR)], s2buf)
        cps = [pltpu.async_copy(xp_hbm.at[s2buf.at[j]],
                                xs.at[pl.ds(j * 128, 128)], sem)
               for j in range(IR)]

        def vbody(k, carry2):
            sv = sbuf[pl.ds(k * 16, 16)] * 4
            dv = dbuf[pl.ds(k * 16, 16)] * 4
            xa = plsc.load_gather(pos4_v, [sv]) - plsc.load_gather(pos4_v, [dv])
            ya = (plsc.load_gather(pos4_v, [sv + 1])
                  - plsc.load_gather(pos4_v, [dv + 1]))
            za = (plsc.load_gather(pos4_v, [sv + 2])
                  - plsc.load_gather(pos4_v, [dv + 2]))
            r2 = xa * xa + ya * ya + za * za + 1e-9
            ir_ = _rsqrt_newton(r2)
            r = r2 * ir_
            ux = xa * ir_
            uy = ya * ir_
            uz = za * ir_
            uz2 = uz * uz
            u = jnp.minimum(jnp.maximum(r * (1.0 / MAX_RADIUS), 0.0), 1.0)
            v = np.pi * (u - 0.5)
            v2 = v * v
            sinv = v * (1.0 + v2 * (-1.0 / 6.0 + v2 * (1.0 / 120.0 + v2 * (
                -1.0 / 5040.0 + v2 * (1.0 / 362880.0)))))
            cut = 0.5 * (1.0 - sinv)
            sh = [None] * 16
            sh[0] = cut
            sh[1] = (s3 * ux) * cut
            sh[2] = (s3 * uy) * cut
            sh[3] = (s3 * uz) * cut
            sh[4] = (s15 * ux) * uy * cut
            sh[5] = (s15 * uy) * uz * cut
            sh[6] = (0.5 * s5) * (3.0 * uz2 - 1.0) * cut
            sh[7] = (s15 * ux) * uz * cut
            sh[8] = (0.5 * s15) * (ux * ux - uy * uy) * cut
            sh[9] = c35 * uy * (3.0 * ux * ux - uy * uy) * cut
            sh[10] = c105 * ux * uy * uz * cut
            sh[11] = c21 * uy * (5.0 * uz2 - 1.0) * cut
            sh[12] = c7 * (5.0 * uz2 - 3.0) * uz * cut
            sh[13] = c21 * ux * (5.0 * uz2 - 1.0) * cut
            sh[14] = (0.5 * c105) * (ux * ux - uy * uy) * uz * cut
            sh[15] = c35 * ux * (ux * ux - uy * uy) * cut
            row = k * 16 + lane
            for f in range(16):
                plsc.store_scatter(geo, [row, jnp.full((16,), f, jnp.int32)],
                                   sh[f])
            for b in range(N_BASIS):
                d = (r - (b * STEP)) * (1.0 / STEP)
                emb = jnp.exp(-(d * d)) * sq10
                plsc.store_scatter(
                    geo, [row, jnp.full((16,), C_EMB + b, jnp.int32)], emb)
            return carry2

        lax.fori_loop(0, CH // 16, vbody, 0)
        for cp in cps:
            cp.wait()
        rows = pl.ds(e0, CH)
        pltpu.sync_copy(geo, pk_out.at[rows, pl.ds(0, 32)])
        pltpu.sync_copy(xs, pk_out.at[rows, pl.ds(C_X, XP)])
        return carry

    nmine = (NCH - wid + NW - 1) // NW
    lax.fori_loop(0, nmine, chunk, 0)


@functools.cache
def _make_sc_geo():
    return pl.kernel(
        _sc_geo_body,
        out_type=jax.ShapeDtypeStruct((E, 128), jnp.float32),
        mesh=plsc.VectorSubcoreMesh(core_axis_name="c", subcore_axis_name="s"),
        scratch_types=[pltpu.VMEM((N * 4,), jnp.float32),
                       pltpu.VMEM((CH,), jnp.int32),
                       pltpu.VMEM((CH,), jnp.int32),
                       pltpu.VMEM((IR, 128), jnp.int32),
                       pltpu.VMEM((CH, 32), jnp.float32),
                       pltpu.VMEM((CH, XP), jnp.float32),
                       pltpu.SemaphoreType.DMA],
        compiler_params=pltpu.CompilerParams(use_tc_tiling_on_sc=False,
                                             needs_layout_passes=False),
    )


def _sc_scatter_body(msg_hbm, dst2_hbm, zeros_hbm, part_out,
                     dbuf, msgb, acc):
    c = lax.axis_index("c")
    s = lax.axis_index("s")
    wid = s * NC + c

    @pl.when(s == 0)
    def _init():
        pltpu.sync_copy(zeros_hbm, acc)

    plsc.subcore_barrier()

    def body(i, carry):
        ci = wid + i * NW
        pltpu.sync_copy(dst2_hbm.at[pl.ds(ci * IR, IR)], dbuf)
        pltpu.sync_copy(msg_hbm.at[pl.ds(ci * CH, CH), pl.ds(0, HA)], msgb)
        for j in range(IR):
            pltpu.sync_copy(msgb.at[pl.ds(j * 128, 128)],
                            acc.at[dbuf.at[j]], add=True)
        return carry

    nmine = (NCH - wid + NW - 1) // NW
    lax.fori_loop(0, nmine, body, 0)

    plsc.subcore_barrier()
    rpt = N // NS
    pltpu.sync_copy(acc.at[pl.ds(s * rpt, rpt)],
                    part_out.at[pl.ds(c * N + s * rpt, rpt)])


@functools.cache
def _make_sc_scatter():
    return pl.kernel(
        _sc_scatter_body,
        out_type=jax.ShapeDtypeStruct((2 * N, HA), jnp.float32),
        mesh=plsc.VectorSubcoreMesh(core_axis_name="c", subcore_axis_name="s"),
        scratch_types=[pltpu.VMEM((IR, 128), jnp.int32),
                       pltpu.VMEM((CH, HA), jnp.float32),
                       pltpu.VMEM_SHARED((N, HA), jnp.float32)],
        compiler_params=pltpu.CompilerParams(use_tc_tiling_on_sc=False),
    )


def _tc_edge_body(pk_ref, W1b_ref, b1_ref, W2_ref, b2_ref, Wshb_ref,
                  Winb_ref, msg_ref):
    pk = pk_ref[...]
    z1 = jnp.maximum(
        jnp.dot(pk, W1b_ref[...], preferred_element_type=jnp.float32)
        + b1_ref[...], 0.0)
    w = jnp.dot(z1, W2_ref[...],
                preferred_element_type=jnp.float32) + b2_ref[...]
    shw = jnp.dot(pk, Wshb_ref[...], preferred_element_type=jnp.float32)
    hs = jnp.dot(pk, Winb_ref[...], preferred_element_type=jnp.float32)
    msg_ref[...] = hs * w * shw


_tc_edge = pl.pallas_call(
    _tc_edge_body,
    grid=(E // BE,),
    in_specs=[
        pl.BlockSpec((BE, 128), lambda i: (i, 0)),
        pl.BlockSpec((128, 128), lambda i: (0, 0)),
        pl.BlockSpec((1, 128), lambda i: (0, 0)),
        pl.BlockSpec((128, HP), lambda i: (0, 0)),
        pl.BlockSpec((1, HP), lambda i: (0, 0)),
        pl.BlockSpec((128, HP), lambda i: (0, 0)),
        pl.BlockSpec((128, HP), lambda i: (0, 0)),
    ],
    out_specs=pl.BlockSpec((BE, HP), lambda i: (i, 0)),
    out_shape=jax.ShapeDtypeStruct((E, HP), jnp.float32),
)


def _tc_final_body(xp_ref, part_ref, batch_ref, Win_ref, Wout_ref, Wlin_ref,
                   blin_ref, out_ref):
    h = jnp.dot(xp_ref[...], Win_ref[...],
                preferred_element_type=jnp.float32)
    part = part_ref[...]
    agg = jnp.concatenate(
        [part[0:N] + part[N:2 * N],
         jnp.zeros((N, HP - HA), jnp.float32)], axis=1)
    h2 = h + agg * INV_SQRT_NEI
    gids = lax.broadcasted_iota(jnp.int32, (G, N), 0).astype(jnp.float32)
    oh = jnp.where(gids == batch_ref[...], 1.0, 0.0)
    sums = jnp.dot(oh, h2, preferred_element_type=jnp.float32)
    counts = jnp.sum(oh, axis=1, keepdims=True)
    pooled = sums / jnp.maximum(counts, 1.0)
    wc = jnp.dot(Wout_ref[...], Wlin_ref[...],
                 preferred_element_type=jnp.float32)
    out_ref[...] = jnp.dot(pooled, wc,
                           preferred_element_type=jnp.float32) + blin_ref[...]


_tc_final = pl.pallas_call(
    _tc_final_body,
    out_shape=jax.ShapeDtypeStruct((G, 128), jnp.float32),
)


def kernel(wt_pos, mt_pos, wt_x, mt_x, wt_batch, mt_batch, edge_index,
           W_in, W1, b1, W2, b2, W_sh, W_out, W_lin, b_lin):
    f32 = jnp.float32
    pos4 = jnp.pad(jnp.concatenate([wt_pos, mt_pos], 0),
                   ((0, 0), (0, 1))).reshape(N * 4)
    x_p = jnp.pad(jnp.concatenate([wt_x, mt_x], 0), ((0, 0), (0, XP - 25)))
    batch = jnp.concatenate([wt_batch, mt_batch]).astype(f32).reshape(1, N)
    ei = edge_index.astype(jnp.int32)
    src1 = ei.reshape(2 * E)
    src2 = ei[0].reshape(E // 128, 128)
    dst2 = ei[1].reshape(E // 128, 128)

    W1b = jnp.zeros((128, 128), f32).at[C_EMB:C_EMB + N_BASIS].set(W1)
    Wshb = jnp.zeros((128, HP), f32).at[0:SH_DIM, 0:H].set(W_sh)
    Winb = jnp.zeros((128, HP), f32).at[C_X:C_X + 25, 0:H].set(W_in)
    W2_p = jnp.pad(W2, ((0, 0), (0, HP - H)))
    b1_r = b1.reshape(1, 128)
    b2_r = jnp.pad(b2, (0, HP - H)).reshape(1, HP)
    Win_p = jnp.pad(W_in, ((0, XP - 25), (0, HP - H)))
    Wout_p = jnp.pad(W_out, ((0, HP - H), (0, 0)))
    Wlin_p = jnp.pad(W_lin, ((0, 0), (0, 128 - 2)))
    blin_p = jnp.pad(b_lin, (0, 128 - 2)).reshape(1, 128)
    zeros_nh = jnp.zeros((N, HA), f32)

    pk = _make_sc_geo()(pos4, x_p, src1, src2)
    msg = _tc_edge(pk, W1b, b1_r, W2_p, b2_r, Wshb, Winb)
    part = _make_sc_scatter()(msg, dst2, zeros_nh)
    outm = _tc_final(x_p, part, batch, Win_p, Wout_p, Wlin_p, blin_p)
    o = outm[:50, :2]
    return (o[:, 0], o[:, 1])


# speedup vs baseline: 6.7085x; 1.0683x over previous
import functools

import jax
import jax.numpy as jnp
import numpy as np
from jax import lax
from jax.experimental import pallas as pl
from jax.experimental.pallas import tpu as pltpu
from jax.experimental.pallas import tpu_sc as plsc

N = 10000
E = 320000
H = 72
HP = 128
XP = 32
G = 64
SH_DIM = 16
N_BASIS = 10
MAX_RADIUS = 20.0
STEP = MAX_RADIUS / (N_BASIS - 1)
INV_SQRT_NEI = float(1.0 / np.sqrt(32.0))

HA = 80
CH = 640
CS = 512
IR = CH // 128
NCH = E // CH
IRS = CS // 128
NCS = E // CS
NC, NS = 2, 16
NW = NC * NS
BE = 2560

C_EMB = 16
C_X = 32


def _rsqrt_newton(r2):
    i = plsc.bitcast(r2, jnp.int32)
    y = plsc.bitcast(jnp.int32(0x5F3759DF) - (i >> 1), jnp.float32)
    for _ in range(3):
        y = y * (1.5 - 0.5 * r2 * y * y)
    return y


def _sc_geo_body(pos4_hbm, xp_hbm, ei2_hbm, pk_out,
                 pos4_v, idxb, geo, xs, sem):
    c = lax.axis_index("c")
    s = lax.axis_index("s")
    wid = s * NC + c
    pltpu.sync_copy(pos4_hbm, pos4_v)

    lane = lax.iota(jnp.int32, 16)
    s3, s5, s15 = np.sqrt(3.0), np.sqrt(5.0), np.sqrt(15.0)
    c35 = np.sqrt(35.0 / 8.0)
    c105 = np.sqrt(105.0)
    c21 = np.sqrt(21.0 / 8.0)
    c7 = 0.5 * np.sqrt(7.0)
    sq10 = np.sqrt(float(N_BASIS))

    def chunk(i, carry):
        ci = wid + i * NW
        e0 = ci * CH
        pltpu.sync_copy(ei2_hbm.at[:, pl.ds(e0, CH)], idxb)
        cps = [pltpu.async_copy(xp_hbm.at[idxb.at[0, pl.ds(j * 128, 128)]],
                                xs.at[pl.ds(j * 128, 128)], sem)
               for j in range(IR)]

        def vbody(k, carry2):
            sv = idxb[0, pl.ds(k * 16, 16)] * 4
            dv = idxb[1, pl.ds(k * 16, 16)] * 4
            xa = plsc.load_gather(pos4_v, [sv]) - plsc.load_gather(pos4_v, [dv])
            ya = (plsc.load_gather(pos4_v, [sv + 1])
                  - plsc.load_gather(pos4_v, [dv + 1]))
            za = (plsc.load_gather(pos4_v, [sv + 2])
                  - plsc.load_gather(pos4_v, [dv + 2]))
            r2 = xa * xa + ya * ya + za * za + 1e-9
            ir_ = _rsqrt_newton(r2)
            r = r2 * ir_
            ux = xa * ir_
            uy = ya * ir_
            uz = za * ir_
            uz2 = uz * uz
            u = jnp.minimum(jnp.maximum(r * (1.0 / MAX_RADIUS), 0.0), 1.0)
            v = np.pi * (u - 0.5)
            v2 = v * v
            sinv = v * (1.0 + v2 * (-1.0 / 6.0 + v2 * (1.0 / 120.0 + v2 * (
                -1.0 / 5040.0 + v2 * (1.0 / 362880.0)))))
            cut = 0.5 * (1.0 - sinv)
            sh = [None] * 16
            sh[0] = cut
            sh[1] = (s3 * ux) * cut
            sh[2] = (s3 * uy) * cut
            sh[3] = (s3 * uz) * cut
            sh[4] = (s15 * ux) * uy * cut
            sh[5] = (s15 * uy) * uz * cut
            sh[6] = (0.5 * s5) * (3.0 * uz2 - 1.0) * cut
            sh[7] = (s15 * ux) * uz * cut
            sh[8] = (0.5 * s15) * (ux * ux - uy * uy) * cut
            sh[9] = c35 * uy * (3.0 * ux * ux - uy * uy) * cut
            sh[10] = c105 * ux * uy * uz * cut
            sh[11] = c21 * uy * (5.0 * uz2 - 1.0) * cut
            sh[12] = c7 * (5.0 * uz2 - 3.0) * uz * cut
            sh[13] = c21 * ux * (5.0 * uz2 - 1.0) * cut
            sh[14] = (0.5 * c105) * (ux * ux - uy * uy) * uz * cut
            sh[15] = c35 * ux * (ux * ux - uy * uy) * cut
            row = k * 16 + lane
            for f in range(16):
                plsc.store_scatter(geo, [row, jnp.full((16,), f, jnp.int32)],
                                   sh[f])
            for b in range(N_BASIS):
                d = (r - (b * STEP)) * (1.0 / STEP)
                emb = jnp.exp(-(d * d)) * sq10
                plsc.store_scatter(
                    geo, [row, jnp.full((16,), C_EMB + b, jnp.int32)], emb)
            return carry2

        lax.fori_loop(0, CH // 16, vbody, 0)
        for cp in cps:
            cp.wait()
        rows = pl.ds(e0, CH)
        pltpu.sync_copy(geo, pk_out.at[rows, pl.ds(0, 32)])
        pltpu.sync_copy(xs, pk_out.at[rows, pl.ds(C_X, XP)])
        return carry

    nmine = (NCH - wid + NW - 1) // NW
    lax.fori_loop(0, nmine, chunk, 0)


@functools.cache
def _make_sc_geo():
    return pl.kernel(
        _sc_geo_body,
        out_type=jax.ShapeDtypeStruct((E, 128), jnp.float32),
        mesh=plsc.VectorSubcoreMesh(core_axis_name="c", subcore_axis_name="s"),
        scratch_types=[pltpu.VMEM((N * 4,), jnp.float32),
                       pltpu.VMEM((2, CH), jnp.int32),
                       pltpu.VMEM((CH, 32), jnp.float32),
                       pltpu.VMEM((CH, XP), jnp.float32),
                       pltpu.SemaphoreType.DMA],
        compiler_params=pltpu.CompilerParams(use_tc_tiling_on_sc=False,
                                             needs_layout_passes=False),
    )


def _sc_scatter_body(msg_hbm, dst2_hbm, zeros_hbm, part_out,
                     dbuf, msgb, acc):
    c = lax.axis_index("c")
    s = lax.axis_index("s")
    wid = s * NC + c

    @pl.when(s == 0)
    def _init():
        pltpu.sync_copy(zeros_hbm, acc)

    plsc.subcore_barrier()

    def body(i, carry):
        ci = wid + i * NW
        pltpu.sync_copy(dst2_hbm.at[pl.ds(ci * IRS, IRS)], dbuf)
        pltpu.sync_copy(msg_hbm.at[pl.ds(ci * CS, CS), pl.ds(0, HA)], msgb)
        for j in range(IRS):
            pltpu.sync_copy(msgb.at[pl.ds(j * 128, 128)],
                            acc.at[dbuf.at[j]], add=True)
        return carry

    nmine = (NCS - wid + NW - 1) // NW
    lax.fori_loop(0, nmine, body, 0)

    plsc.subcore_barrier()
    rpt = N // NS
    pltpu.sync_copy(acc.at[pl.ds(s * rpt, rpt)],
                    part_out.at[pl.ds(c * N + s * rpt, rpt)])


@functools.cache
def _make_sc_scatter():
    return pl.kernel(
        _sc_scatter_body,
        out_type=jax.ShapeDtypeStruct((2 * N, HA), jnp.float32),
        mesh=plsc.VectorSubcoreMesh(core_axis_name="c", subcore_axis_name="s"),
        scratch_types=[pltpu.VMEM((IRS, 128), jnp.int32),
                       pltpu.VMEM((CS, HA), jnp.float32),
                       pltpu.VMEM_SHARED((N, HA), jnp.float32)],
        compiler_params=pltpu.CompilerParams(use_tc_tiling_on_sc=False),
    )


def _tc_edge_body(pk_ref, W1b_ref, b1_ref, W2_ref, b2_ref, Wshb_ref,
                  Winb_ref, msg_ref):
    pk = pk_ref[...]
    z1 = jnp.maximum(
        jnp.dot(pk, W1b_ref[...], preferred_element_type=jnp.float32)
        + b1_ref[...], 0.0)
    w = jnp.dot(z1, W2_ref[...],
                preferred_element_type=jnp.float32) + b2_ref[...]
    shw = jnp.dot(pk, Wshb_ref[...], preferred_element_type=jnp.float32)
    hs = jnp.dot(pk, Winb_ref[...], preferred_element_type=jnp.float32)
    msg_ref[...] = hs * w * shw


_tc_edge = pl.pallas_call(
    _tc_edge_body,
    grid=(E // BE,),
    in_specs=[
        pl.BlockSpec((BE, 128), lambda i: (i, 0)),
        pl.BlockSpec((128, 128), lambda i: (0, 0)),
        pl.BlockSpec((1, 128), lambda i: (0, 0)),
        pl.BlockSpec((128, HP), lambda i: (0, 0)),
        pl.BlockSpec((1, HP), lambda i: (0, 0)),
        pl.BlockSpec((128, HP), lambda i: (0, 0)),
        pl.BlockSpec((128, HP), lambda i: (0, 0)),
    ],
    out_specs=pl.BlockSpec((BE, HP), lambda i: (i, 0)),
    out_shape=jax.ShapeDtypeStruct((E, HP), jnp.float32),
)


def _tc_final_body(xp_ref, part_ref, batch_ref, Win_ref, Wout_ref, Wlin_ref,
                   blin_ref, out_ref):
    h = jnp.dot(xp_ref[...], Win_ref[...],
                preferred_element_type=jnp.float32)
    part = part_ref[...]
    agg = jnp.concatenate(
        [part[0:N] + part[N:2 * N],
         jnp.zeros((N, HP - HA), jnp.float32)], axis=1)
    h2 = h + agg * INV_SQRT_NEI
    gids = lax.broadcasted_iota(jnp.int32, (G, N), 0).astype(jnp.float32)
    oh = jnp.where(gids == batch_ref[...], 1.0, 0.0)
    sums = jnp.dot(oh, h2, preferred_element_type=jnp.float32)
    counts = jnp.sum(oh, axis=1, keepdims=True)
    pooled = sums / jnp.maximum(counts, 1.0)
    wc = jnp.dot(Wout_ref[...], Wlin_ref[...],
                 preferred_element_type=jnp.float32)
    out_ref[...] = jnp.dot(pooled, wc,
                           preferred_element_type=jnp.float32) + blin_ref[...]


_tc_final = pl.pallas_call(
    _tc_final_body,
    out_shape=jax.ShapeDtypeStruct((G, 128), jnp.float32),
)


def kernel(wt_pos, mt_pos, wt_x, mt_x, wt_batch, mt_batch, edge_index,
           W_in, W1, b1, W2, b2, W_sh, W_out, W_lin, b_lin):
    f32 = jnp.float32
    pos4 = jnp.pad(jnp.concatenate([wt_pos, mt_pos], 0),
                   ((0, 0), (0, 1))).reshape(N * 4)
    x_p = jnp.pad(jnp.concatenate([wt_x, mt_x], 0), ((0, 0), (0, XP - 25)))
    batch = jnp.concatenate([wt_batch, mt_batch]).astype(f32).reshape(1, N)
    ei = edge_index.astype(jnp.int32)
    ei2 = ei
    dst2 = ei[1].reshape(E // 128, 128)

    W1b = jnp.zeros((128, 128), f32).at[C_EMB:C_EMB + N_BASIS].set(W1)
    Wshb = jnp.zeros((128, HP), f32).at[0:SH_DIM, 0:H].set(W_sh)
    Winb = jnp.zeros((128, HP), f32).at[C_X:C_X + 25, 0:H].set(W_in)
    W2_p = jnp.pad(W2, ((0, 0), (0, HP - H)))
    b1_r = b1.reshape(1, 128)
    b2_r = jnp.pad(b2, (0, HP - H)).reshape(1, HP)
    Win_p = jnp.pad(W_in, ((0, XP - 25), (0, HP - H)))
    Wout_p = jnp.pad(W_out, ((0, HP - H), (0, 0)))
    Wlin_p = jnp.pad(W_lin, ((0, 0), (0, 128 - 2)))
    blin_p = jnp.pad(b_lin, (0, 128 - 2)).reshape(1, 128)
    zeros_nh = jnp.zeros((N, HA), f32)

    pk = _make_sc_geo()(pos4, x_p, ei2)
    msg = _tc_edge(pk, W1b, b1_r, W2_p, b2_r, Wshb, Winb)
    part = _make_sc_scatter()(msg, dst2, zeros_nh)
    outm = _tc_final(x_p, part, batch, Win_p, Wout_p, Wlin_p, blin_p)
    o = outm[:50, :2]
    return (o[:, 0], o[:, 1])


# speedup vs baseline: 7.3603x; 1.0972x over previous
import functools

import jax
import jax.numpy as jnp
import numpy as np
from jax import lax
from jax.experimental import pallas as pl
from jax.experimental.pallas import tpu as pltpu
from jax.experimental.pallas import tpu_sc as plsc

N = 10000
E = 320000
H = 72
HP = 128
XP = 32
G = 64
SH_DIM = 16
N_BASIS = 10
MAX_RADIUS = 20.0
STEP = MAX_RADIUS / (N_BASIS - 1)
INV_SQRT_NEI = float(1.0 / np.sqrt(32.0))

HA = 80
CH = 640
CS = 512
IR = CH // 128
NCH = E // CH
IRS = CS // 128
NCS = E // CS
NC, NS = 2, 16
NW = NC * NS
BE = 6400

C_EMB = 16
C_X = 32


def _rsqrt_newton(r2):
    i = plsc.bitcast(r2, jnp.int32)
    y = plsc.bitcast(jnp.int32(0x5F3759DF) - (i >> 1), jnp.float32)
    for _ in range(3):
        y = y * (1.5 - 0.5 * r2 * y * y)
    return y


def _sc_geo_body(pos4_hbm, xp_hbm, ei2_hbm, pk_out,
                 pos4_v, idxb, geo, xs, sem):
    c = lax.axis_index("c")
    s = lax.axis_index("s")
    wid = s * NC + c
    pltpu.sync_copy(pos4_hbm, pos4_v)

    lane = lax.iota(jnp.int32, 16)
    s3, s5, s15 = np.sqrt(3.0), np.sqrt(5.0), np.sqrt(15.0)
    c35 = np.sqrt(35.0 / 8.0)
    c105 = np.sqrt(105.0)
    c21 = np.sqrt(21.0 / 8.0)
    c7 = 0.5 * np.sqrt(7.0)
    sq10 = np.sqrt(float(N_BASIS))

    def chunk(i, carry):
        ci = wid + i * NW
        e0 = ci * CH
        pltpu.sync_copy(ei2_hbm.at[:, pl.ds(e0, CH)], idxb)
        cps = [pltpu.async_copy(xp_hbm.at[idxb.at[0, pl.ds(j * 128, 128)]],
                                xs.at[pl.ds(j * 128, 128)], sem)
               for j in range(IR)]

        def vbody(k, carry2):
            sv = idxb[0, pl.ds(k * 16, 16)] * 4
            dv = idxb[1, pl.ds(k * 16, 16)] * 4
            xa = plsc.load_gather(pos4_v, [sv]) - plsc.load_gather(pos4_v, [dv])
            ya = (plsc.load_gather(pos4_v, [sv + 1])
                  - plsc.load_gather(pos4_v, [dv + 1]))
            za = (plsc.load_gather(pos4_v, [sv + 2])
                  - plsc.load_gather(pos4_v, [dv + 2]))
            r2 = xa * xa + ya * ya + za * za + 1e-9
            ir_ = _rsqrt_newton(r2)
            r = r2 * ir_
            ux = xa * ir_
            uy = ya * ir_
            uz = za * ir_
            uz2 = uz * uz
            u = jnp.minimum(jnp.maximum(r * (1.0 / MAX_RADIUS), 0.0), 1.0)
            v = np.pi * (u - 0.5)
            v2 = v * v
            sinv = v * (1.0 + v2 * (-1.0 / 6.0 + v2 * (1.0 / 120.0 + v2 * (
                -1.0 / 5040.0 + v2 * (1.0 / 362880.0)))))
            cut = 0.5 * (1.0 - sinv)
            sh = [None] * 16
            sh[0] = cut
            sh[1] = (s3 * ux) * cut
            sh[2] = (s3 * uy) * cut
            sh[3] = (s3 * uz) * cut
            sh[4] = (s15 * ux) * uy * cut
            sh[5] = (s15 * uy) * uz * cut
            sh[6] = (0.5 * s5) * (3.0 * uz2 - 1.0) * cut
            sh[7] = (s15 * ux) * uz * cut
            sh[8] = (0.5 * s15) * (ux * ux - uy * uy) * cut
            sh[9] = c35 * uy * (3.0 * ux * ux - uy * uy) * cut
            sh[10] = c105 * ux * uy * uz * cut
            sh[11] = c21 * uy * (5.0 * uz2 - 1.0) * cut
            sh[12] = c7 * (5.0 * uz2 - 3.0) * uz * cut
            sh[13] = c21 * ux * (5.0 * uz2 - 1.0) * cut
            sh[14] = (0.5 * c105) * (ux * ux - uy * uy) * uz * cut
            sh[15] = c35 * ux * (ux * ux - uy * uy) * cut
            row = k * 16 + lane
            for f in range(16):
                plsc.store_scatter(geo, [row, jnp.full((16,), f, jnp.int32)],
                                   sh[f])
            for b in range(N_BASIS):
                d = (r - (b * STEP)) * (1.0 / STEP)
                emb = jnp.exp(-(d * d)) * sq10
                plsc.store_scatter(
                    geo, [row, jnp.full((16,), C_EMB + b, jnp.int32)], emb)
            return carry2

        lax.fori_loop(0, CH // 16, vbody, 0)
        for cp in cps:
            cp.wait()
        rows = pl.ds(e0, CH)
        pltpu.sync_copy(geo, pk_out.at[rows, pl.ds(0, 32)])
        pltpu.sync_copy(xs, pk_out.at[rows, pl.ds(C_X, XP)])
        return carry

    nmine = (NCH - wid + NW - 1) // NW
    lax.fori_loop(0, nmine, chunk, 0)


@functools.cache
def _make_sc_geo():
    return pl.kernel(
        _sc_geo_body,
        out_type=jax.ShapeDtypeStruct((E, 128), jnp.float32),
        mesh=plsc.VectorSubcoreMesh(core_axis_name="c", subcore_axis_name="s"),
        scratch_types=[pltpu.VMEM((N * 4,), jnp.float32),
                       pltpu.VMEM((2, CH), jnp.int32),
                       pltpu.VMEM((CH, 32), jnp.float32),
                       pltpu.VMEM((CH, XP), jnp.float32),
                       pltpu.SemaphoreType.DMA],
        compiler_params=pltpu.CompilerParams(use_tc_tiling_on_sc=False,
                                             needs_layout_passes=False),
    )


def _sc_scatter_body(msg_hbm, dst2_hbm, zeros_hbm, part_out,
                     dbuf, msgb, acc):
    c = lax.axis_index("c")
    s = lax.axis_index("s")
    wid = s * NC + c

    @pl.when(s == 0)
    def _init():
        pltpu.sync_copy(zeros_hbm, acc)

    plsc.subcore_barrier()

    def body(i, carry):
        ci = wid + i * NW
        pltpu.sync_copy(dst2_hbm.at[pl.ds(ci * IRS, IRS)], dbuf)
        pltpu.sync_copy(msg_hbm.at[pl.ds(ci * CS, CS), pl.ds(0, HA)], msgb)
        for j in range(IRS):
            pltpu.sync_copy(msgb.at[pl.ds(j * 128, 128)],
                            acc.at[dbuf.at[j]], add=True)
        return carry

    nmine = (NCS - wid + NW - 1) // NW
    lax.fori_loop(0, nmine, body, 0)

    plsc.subcore_barrier()
    rpt = N // NS
    pltpu.sync_copy(acc.at[pl.ds(s * rpt, rpt)],
                    part_out.at[pl.ds(c * N + s * rpt, rpt)])


@functools.cache
def _make_sc_scatter():
    return pl.kernel(
        _sc_scatter_body,
        out_type=jax.ShapeDtypeStruct((2 * N, HA), jnp.float32),
        mesh=plsc.VectorSubcoreMesh(core_axis_name="c", subcore_axis_name="s"),
        scratch_types=[pltpu.VMEM((IRS, 128), jnp.int32),
                       pltpu.VMEM((CS, HA), jnp.float32),
                       pltpu.VMEM_SHARED((N, HA), jnp.float32)],
        compiler_params=pltpu.CompilerParams(use_tc_tiling_on_sc=False),
    )


def _tc_edge_body(pk_ref, W1b_ref, b1_ref, W2_ref, b2_ref, Wshb_ref,
                  Winb_ref, msg_ref):
    pk = pk_ref[...]
    z1 = jnp.maximum(
        jnp.dot(pk, W1b_ref[...], preferred_element_type=jnp.float32)
        + b1_ref[...], 0.0)
    w = jnp.dot(z1, W2_ref[...],
                preferred_element_type=jnp.float32) + b2_ref[...]
    shw = jnp.dot(pk, Wshb_ref[...], preferred_element_type=jnp.float32)
    hs = jnp.dot(pk, Winb_ref[...], preferred_element_type=jnp.float32)
    msg_ref[...] = hs * w * shw


_tc_edge = pl.pallas_call(
    _tc_edge_body,
    grid=(E // BE,),
    in_specs=[
        pl.BlockSpec((BE, 128), lambda i: (i, 0)),
        pl.BlockSpec((128, 128), lambda i: (0, 0)),
        pl.BlockSpec((1, 128), lambda i: (0, 0)),
        pl.BlockSpec((128, HP), lambda i: (0, 0)),
        pl.BlockSpec((1, HP), lambda i: (0, 0)),
        pl.BlockSpec((128, HP), lambda i: (0, 0)),
        pl.BlockSpec((128, HP), lambda i: (0, 0)),
    ],
    out_specs=pl.BlockSpec((BE, HP), lambda i: (i, 0)),
    out_shape=jax.ShapeDtypeStruct((E, HP), jnp.float32),
)


def _tc_final_body(xp_ref, part_ref, batch_ref, Win_ref, Wout_ref, Wlin_ref,
                   blin_ref, out_ref):
    h = jnp.dot(xp_ref[...], Win_ref[...],
                preferred_element_type=jnp.float32)
    part = part_ref[...]
    agg = jnp.concatenate(
        [part[0:N] + part[N:2 * N],
         jnp.zeros((N, HP - HA), jnp.float32)], axis=1)
    h2 = h + agg * INV_SQRT_NEI
    gids = lax.broadcasted_iota(jnp.int32, (G, N), 0).astype(jnp.float32)
    oh = jnp.where(gids == batch_ref[...], 1.0, 0.0)
    sums = jnp.dot(oh, h2, preferred_element_type=jnp.float32)
    counts = jnp.sum(oh, axis=1, keepdims=True)
    pooled = sums / jnp.maximum(counts, 1.0)
    wc = jnp.dot(Wout_ref[...], Wlin_ref[...],
                 preferred_element_type=jnp.float32)
    out_ref[...] = jnp.dot(pooled, wc,
                           preferred_element_type=jnp.float32) + blin_ref[...]


_tc_final = pl.pallas_call(
    _tc_final_body,
    out_shape=jax.ShapeDtypeStruct((G, 128), jnp.float32),
)


def kernel(wt_pos, mt_pos, wt_x, mt_x, wt_batch, mt_batch, edge_index,
           W_in, W1, b1, W2, b2, W_sh, W_out, W_lin, b_lin):
    f32 = jnp.float32
    pos4 = jnp.pad(jnp.concatenate([wt_pos, mt_pos], 0),
                   ((0, 0), (0, 1))).reshape(N * 4)
    x_p = jnp.pad(jnp.concatenate([wt_x, mt_x], 0), ((0, 0), (0, XP - 25)))
    batch = jnp.concatenate([wt_batch, mt_batch]).astype(f32).reshape(1, N)
    ei = edge_index.astype(jnp.int32)
    ei2 = ei
    dst2 = ei[1].reshape(E // 128, 128)

    W1b = jnp.zeros((128, 128), f32).at[C_EMB:C_EMB + N_BASIS].set(W1)
    Wshb = jnp.zeros((128, HP), f32).at[0:SH_DIM, 0:H].set(W_sh)
    Winb = jnp.zeros((128, HP), f32).at[C_X:C_X + 25, 0:H].set(W_in)
    W2_p = jnp.pad(W2, ((0, 0), (0, HP - H)))
    b1_r = b1.reshape(1, 128)
    b2_r = jnp.pad(b2, (0, HP - H)).reshape(1, HP)
    Win_p = jnp.pad(W_in, ((0, XP - 25), (0, HP - H)))
    Wout_p = jnp.pad(W_out, ((0, HP - H), (0, 0)))
    Wlin_p = jnp.pad(W_lin, ((0, 0), (0, 128 - 2)))
    blin_p = jnp.pad(b_lin, (0, 128 - 2)).reshape(1, 128)
    zeros_nh = jnp.zeros((N, HA), f32)

    pk = _make_sc_geo()(pos4, x_p, ei2)
    msg = _tc_edge(pk, W1b, b1_r, W2_p, b2_r, Wshb, Winb)
    part = _make_sc_scatter()(msg, dst2, zeros_nh)
    outm = _tc_final(x_p, part, batch, Win_p, Wout_p, Wlin_p, blin_p)
    o = outm[:50, :2]
    return (o[:, 0], o[:, 1])
